# baseline (speedup 1.0000x reference)
import functools

import jax
import jax.numpy as jnp
from jax.experimental import pallas as pl
from jax.experimental.pallas import tpu as pltpu


def _fused_kernel(n_tok, idx_ref, tgt_ref, row_ref, out_ref, loss_ref):
    i = pl.program_id(0)
    row = row_ref[0]
    out_ref[0] = row
    m = jnp.max(row)
    s = jnp.sum(jnp.exp(row - m))
    logz = m + jnp.log(s)
    t = tgt_ref[i]
    cols = jax.lax.broadcasted_iota(jnp.int32, row.shape, 1)
    picked = jnp.sum(jnp.where(cols == t, row, 0.0))

    @pl.when(i == 0)
    def _init():
        loss_ref[...] = jnp.zeros((1, 1), jnp.float32)

    loss_ref[...] += jnp.reshape(logz - picked, (1, 1))

    @pl.when(i == n_tok - 1)
    def _finish():
        loss_ref[...] = loss_ref[...] / n_tok


def kernel(idx, targets, table):
    B, T = idx.shape
    V, C = table.shape
    n_tok = B * T
    idx_flat = idx.reshape(n_tok).astype(jnp.int32)
    tgt_flat = targets.reshape(n_tok).astype(jnp.int32)

    grid_spec = pltpu.PrefetchScalarGridSpec(
        num_scalar_prefetch=2,
        grid=(n_tok,),
        in_specs=[
            pl.BlockSpec((1, 1, C), lambda i, idx_ref, tgt_ref: (idx_ref[i], 0, 0)),
        ],
        out_specs=[
            pl.BlockSpec((1, 1, C), lambda i, idx_ref, tgt_ref: (i, 0, 0)),
            pl.BlockSpec((1, 1), lambda i, idx_ref, tgt_ref: (0, 0)),
        ],
    )
    logits_flat, loss = pl.pallas_call(
        functools.partial(_fused_kernel, n_tok),
        grid_spec=grid_spec,
        out_shape=[
            jax.ShapeDtypeStruct((n_tok, 1, C), jnp.float32),
            jax.ShapeDtypeStruct((1, 1), jnp.float32),
        ],
    )(idx_flat, tgt_flat, table.reshape(V, 1, C))
    return logits_flat.reshape(B, T, C), loss[0, 0]


# speedup vs baseline: 12.4738x; 12.4738x over previous
import functools

import jax
import jax.numpy as jnp
from jax.experimental import pallas as pl
from jax.experimental.pallas import tpu as pltpu

_CH = 16


def _fused_kernel(n_steps, n_tok, idx_ref, tgt_ref, table_ref, out_ref,
                  loss_ref, rows, sems):
    i = pl.program_id(0)
    slot = jax.lax.rem(i, 2)

    @pl.when(i == 0)
    def _prime():
        for j in range(_CH):
            pltpu.make_async_copy(
                table_ref.at[idx_ref[j]], rows.at[0, j], sems.at[0, j]
            ).start()

    @pl.when(i + 1 < n_steps)
    def _prefetch_next():
        nxt = jax.lax.rem(i + 1, 2)
        for j in range(_CH):
            pltpu.make_async_copy(
                table_ref.at[idx_ref[(i + 1) * _CH + j]],
                rows.at[nxt, j],
                sems.at[nxt, j],
            ).start()

    for j in range(_CH):
        pltpu.make_async_copy(
            table_ref.at[idx_ref[i * _CH + j]], rows.at[slot, j],
            sems.at[slot, j]
        ).wait()

    block = rows[slot]
    out_ref[...] = block
    m = jnp.max(block, axis=1, keepdims=True)
    e = jnp.exp(block - m)
    s = jnp.sum(e, axis=1, keepdims=True)
    logz = m + jnp.log(s)
    t = jnp.stack([tgt_ref[i * _CH + j] for j in range(_CH)])
    cols = jax.lax.broadcasted_iota(jnp.int32, block.shape, 1)
    picked = jnp.sum(
        jnp.where(cols == t[:, None], block, 0.0), axis=1, keepdims=True
    )
    part = jnp.sum(logz - picked)

    @pl.when(i == 0)
    def _init():
        loss_ref[...] = jnp.zeros((1, 1), jnp.float32)

    loss_ref[...] += part

    @pl.when(i == n_steps - 1)
    def _finish():
        loss_ref[...] = loss_ref[...] / n_tok


def kernel(idx, targets, table):
    B, T = idx.shape
    V, C = table.shape
    n_tok = B * T
    n_steps = n_tok // _CH
    idx_flat = idx.reshape(n_tok).astype(jnp.int32)
    tgt_flat = targets.reshape(n_tok).astype(jnp.int32)

    grid_spec = pltpu.PrefetchScalarGridSpec(
        num_scalar_prefetch=2,
        grid=(n_steps,),
        in_specs=[
            pl.BlockSpec(memory_space=pltpu.HBM),
        ],
        out_specs=[
            pl.BlockSpec((_CH, C), lambda i, idx_ref, tgt_ref: (i, 0)),
            pl.BlockSpec((1, 1), lambda i, idx_ref, tgt_ref: (0, 0)),
        ],
        scratch_shapes=[
            pltpu.VMEM((2, _CH, C), jnp.float32),
            pltpu.SemaphoreType.DMA((2, _CH)),
        ],
    )
    logits_flat, loss = pl.pallas_call(
        functools.partial(_fused_kernel, n_steps, n_tok),
        grid_spec=grid_spec,
        out_shape=[
            jax.ShapeDtypeStruct((n_tok, C), jnp.float32),
            jax.ShapeDtypeStruct((1, 1), jnp.float32),
        ],
    )(idx_flat, tgt_flat, table)
    return logits_flat.reshape(B, T, C), loss[0, 0]
